# baseline (speedup 1.0000x reference)
import functools

import jax
import jax.numpy as jnp
from jax.experimental import pallas as pl
from jax.experimental.pallas import tpu as pltpu

_B_LANES = 256


def _net_kernel(x_ref, w1_ref, b1_ref, w2_ref, b2_ref, w3_ref, b3_ref,
                f1_ref, fb1_ref, f2_ref, fb2_ref, o_ref):
    xb = x_ref[...]

    rows = []
    for oh in range(20):
        slab = xb[4 * oh:4 * oh + 8].reshape(8 * 84 * 4, _B_LANES)
        acc = jnp.dot(w1_ref[...], slab, preferred_element_type=jnp.float32)
        rows.append(jnp.maximum(acc + b1_ref[...], 0.0).astype(jnp.bfloat16))
    a = jnp.stack(rows)

    rows = []
    for oh in range(9):
        slab = a[2 * oh:2 * oh + 4].reshape(4 * 640, _B_LANES)
        acc = jnp.dot(w2_ref[...], slab, preferred_element_type=jnp.float32)
        rows.append(jnp.maximum(acc + b2_ref[...], 0.0).astype(jnp.bfloat16))
    a = jnp.stack(rows)

    rows = []
    for oh in range(7):
        slab = a[oh:oh + 3].reshape(3 * 576, _B_LANES)
        acc = jnp.dot(w3_ref[...], slab, preferred_element_type=jnp.float32)
        rows.append(jnp.maximum(acc + b3_ref[...], 0.0).astype(jnp.bfloat16))
    feat = jnp.stack(rows).reshape(7 * 448, _B_LANES)

    h = jax.lax.dot_general(f1_ref[...], feat, (((0,), (0,)), ((), ())),
                            preferred_element_type=jnp.float32)
    h = jnp.maximum(h + fb1_ref[...], 0.0).astype(jnp.bfloat16)
    o = jax.lax.dot_general(f2_ref[...], h, (((0,), (0,)), ((), ())),
                            preferred_element_type=jnp.float32)
    o_ref[...] = o + fb2_ref[...]


def _sel_weight_lhs(w_ijco, n_in, n_out, stride):
    kh, k, C, O = w_ijco.shape
    base = jnp.transpose(w_ijco, (3, 0, 1, 2))
    base = jnp.pad(base, ((0, 0), (0, 0), (0, n_in - k), (0, 0)))
    base = base.reshape(O, kh * n_in * C).astype(jnp.bfloat16)
    rows = jnp.stack([jnp.roll(base, stride * C * t, axis=1)
                      for t in range(n_out)])
    return rows.reshape(n_out * O, kh * n_in * C)


def kernel(c1_w, c1_b, c2_w, c2_b, c3_w, c3_b, fc1_w, fc1_b, fc2_w, fc2_b,
           x_nchw):
    N = x_nchw.shape[0]
    xt = jnp.transpose(x_nchw, (2, 3, 1, 0)).astype(jnp.bfloat16)

    w1 = _sel_weight_lhs(c1_w.reshape(8, 8, 4, 32), 84, 20, 4)
    w2 = _sel_weight_lhs(c2_w.reshape(4, 4, 32, 64), 20, 9, 2)
    w3 = _sel_weight_lhs(c3_w.reshape(3, 3, 64, 64), 9, 7, 1)
    b1 = jnp.tile(c1_b.reshape(-1), 20).reshape(640, 1)
    b2 = jnp.tile(c2_b.reshape(-1), 9).reshape(576, 1)
    b3 = jnp.tile(c3_b.reshape(-1), 7).reshape(448, 1)
    fb1 = fc1_b.reshape(512, 1)
    fb2 = fc2_b.reshape(18, 1)

    out = pl.pallas_call(
        _net_kernel,
        out_shape=jax.ShapeDtypeStruct((18, N), jnp.float32),
        grid=(N // _B_LANES,),
        in_specs=[
            pl.BlockSpec((84, 84, 4, _B_LANES), lambda g: (0, 0, 0, g)),
            pl.BlockSpec((640, 2688), lambda g: (0, 0)),
            pl.BlockSpec((640, 1), lambda g: (0, 0)),
            pl.BlockSpec((576, 2560), lambda g: (0, 0)),
            pl.BlockSpec((576, 1), lambda g: (0, 0)),
            pl.BlockSpec((448, 1728), lambda g: (0, 0)),
            pl.BlockSpec((448, 1), lambda g: (0, 0)),
            pl.BlockSpec((3136, 512), lambda g: (0, 0)),
            pl.BlockSpec((512, 1), lambda g: (0, 0)),
            pl.BlockSpec((512, 18), lambda g: (0, 0)),
            pl.BlockSpec((18, 1), lambda g: (0, 0)),
        ],
        out_specs=pl.BlockSpec((18, _B_LANES), lambda g: (0, g)),
        compiler_params=pltpu.CompilerParams(
            dimension_semantics=("arbitrary",)),
    )(xt, w1, b1, w2, b2, w3, b3,
      fc1_w.astype(jnp.bfloat16), fb1, fc2_w.astype(jnp.bfloat16), fb2)

    return out.T


# speedup vs baseline: 1.6505x; 1.6505x over previous
import functools

import jax
import jax.numpy as jnp
from jax.experimental import pallas as pl
from jax.experimental.pallas import tpu as pltpu

_B_LANES = 256


def _net_kernel(x_ref, b1d_ref, b1_ref, b2d_ref, b2_ref, b3d_ref, b3_ref,
                f1_ref, fb1_ref, f2_ref, fb2_ref, o_ref,
                w1_s, w2_s, w3_s):
    @pl.when(pl.program_id(0) == 0)
    def _build():
        for t in range(20):
            w1_s[32 * t:32 * t + 32, :] = b1d_ref[:, 2688 - 16 * t:
                                                  5376 - 16 * t]
        for t in range(9):
            w2_s[64 * t:64 * t + 64, :] = b2d_ref[:, 2560 - 64 * t:
                                                  5120 - 64 * t]
        for t in range(7):
            w3_s[64 * t:64 * t + 64, :] = b3d_ref[:, 1728 - 64 * t:
                                                  3456 - 64 * t]

    xb = x_ref[...]

    w1 = w1_s[...]
    rows = []
    for oh in range(20):
        slab = xb[4 * oh:4 * oh + 8].reshape(8 * 84 * 4, _B_LANES)
        acc = jnp.dot(w1, slab, preferred_element_type=jnp.float32)
        rows.append(jnp.maximum(acc + b1_ref[...], 0.0).astype(jnp.bfloat16))
    a = jnp.stack(rows)

    w2 = w2_s[...]
    rows = []
    for oh in range(9):
        slab = a[2 * oh:2 * oh + 4].reshape(4 * 640, _B_LANES)
        acc = jnp.dot(w2, slab, preferred_element_type=jnp.float32)
        rows.append(jnp.maximum(acc + b2_ref[...], 0.0).astype(jnp.bfloat16))
    a = jnp.stack(rows)

    w3 = w3_s[...]
    rows = []
    for oh in range(7):
        slab = a[oh:oh + 3].reshape(3 * 576, _B_LANES)
        acc = jnp.dot(w3, slab, preferred_element_type=jnp.float32)
        rows.append(jnp.maximum(acc + b3_ref[...], 0.0).astype(jnp.bfloat16))
    feat = jnp.stack(rows).reshape(7 * 448, _B_LANES)

    h = jax.lax.dot_general(f1_ref[...], feat, (((0,), (0,)), ((), ())),
                            preferred_element_type=jnp.float32)
    h = jnp.maximum(h + fb1_ref[...], 0.0).astype(jnp.bfloat16)
    o = jax.lax.dot_general(f2_ref[...], h, (((0,), (0,)), ((), ())),
                            preferred_element_type=jnp.float32)
    o_ref[...] = o + fb2_ref[...]


def _base_doubled(w_ijco, n_in):
    kh, k, C, O = w_ijco.shape
    base = jnp.transpose(w_ijco, (3, 0, 1, 2))
    base = jnp.pad(base, ((0, 0), (0, 0), (0, n_in - k), (0, 0)))
    base = base.reshape(O, kh * n_in * C).astype(jnp.bfloat16)
    return jnp.concatenate([base, base], axis=1)


def kernel(c1_w, c1_b, c2_w, c2_b, c3_w, c3_b, fc1_w, fc1_b, fc2_w, fc2_b,
           x_nchw):
    N = x_nchw.shape[0]
    xt = jnp.transpose(x_nchw, (2, 3, 1, 0)).astype(jnp.bfloat16)

    b1d = _base_doubled(c1_w.reshape(8, 8, 4, 32), 84)
    b2d = _base_doubled(c2_w.reshape(4, 4, 32, 64), 20)
    b3d = _base_doubled(c3_w.reshape(3, 3, 64, 64), 9)
    b1 = jnp.tile(c1_b.reshape(-1), 20).reshape(640, 1)
    b2 = jnp.tile(c2_b.reshape(-1), 9).reshape(576, 1)
    b3 = jnp.tile(c3_b.reshape(-1), 7).reshape(448, 1)
    fb1 = fc1_b.reshape(512, 1)
    fb2 = fc2_b.reshape(18, 1)

    out = pl.pallas_call(
        _net_kernel,
        out_shape=jax.ShapeDtypeStruct((18, N), jnp.float32),
        grid=(N // _B_LANES,),
        in_specs=[
            pl.BlockSpec((84, 84, 4, _B_LANES), lambda g: (0, 0, 0, g)),
            pl.BlockSpec((32, 5376), lambda g: (0, 0)),
            pl.BlockSpec((640, 1), lambda g: (0, 0)),
            pl.BlockSpec((64, 5120), lambda g: (0, 0)),
            pl.BlockSpec((576, 1), lambda g: (0, 0)),
            pl.BlockSpec((64, 3456), lambda g: (0, 0)),
            pl.BlockSpec((448, 1), lambda g: (0, 0)),
            pl.BlockSpec((3136, 512), lambda g: (0, 0)),
            pl.BlockSpec((512, 1), lambda g: (0, 0)),
            pl.BlockSpec((512, 18), lambda g: (0, 0)),
            pl.BlockSpec((18, 1), lambda g: (0, 0)),
        ],
        out_specs=pl.BlockSpec((18, _B_LANES), lambda g: (0, g)),
        scratch_shapes=[
            pltpu.VMEM((640, 2688), jnp.bfloat16),
            pltpu.VMEM((576, 2560), jnp.bfloat16),
            pltpu.VMEM((448, 1728), jnp.bfloat16),
        ],
        compiler_params=pltpu.CompilerParams(
            dimension_semantics=("arbitrary",)),
    )(xt, b1d, b1, b2d, b2, b3d, b3,
      fc1_w.astype(jnp.bfloat16), fb1, fc2_w.astype(jnp.bfloat16), fb2)

    return out.T
